# baseline (speedup 1.0000x reference)
import functools

import jax
import jax.numpy as jnp
from jax import lax
from jax.experimental import pallas as pl
from jax.experimental.pallas import tpu as pltpu
from jax.experimental.pallas import tpu_sc as plsc

B = 4
S = 2048
D = 1024
L = 16
NL = D // L

NC = 2
NS = 16
NW = NC * NS
S_PER_W = S // NW
CH = 16
NCHUNK = S_PER_W // CH
NU = B * NCHUNK
NB = 5
NPB = 2


def _make_kernel():
    mesh = plsc.VectorSubcoreMesh(core_axis_name="c", subcore_axis_name="s")

    @functools.partial(
        pl.kernel,
        mesh=mesh,
        out_type=jax.ShapeDtypeStruct((B, S, D), jnp.float32),
        scratch_types=[
            pltpu.VMEM((NU * CH,), jnp.int32),
            pltpu.VMEM((NPB, CH, D), jnp.float32),
            pltpu.VMEM((NB, CH, D), jnp.float32),
        ]
        + [pltpu.SemaphoreType.DMA] * (2 * NB + NPB),
    )
    def emb_kernel(tok_hbm, tab_hbm, pos_hbm, out_hbm, idx_v, pos_v, rows_v, *sems):
        gsem, ssem, psem = sems[:NB], sems[NB : 2 * NB], sems[2 * NB :]
        wid = lax.axis_index("s") * NC + lax.axis_index("c")
        s0 = wid * S_PER_W

        pltpu.sync_copy(tok_hbm.at[wid], idx_v)

        def start_gather(u):
            return pltpu.async_copy(
                tab_hbm.at[idx_v.at[pl.ds(u * CH, CH)]],
                rows_v.at[u % NB],
                gsem[u % NB],
            )

        def start_pos(j):
            return pltpu.async_copy(
                pos_hbm.at[pl.ds(s0 + j * CH, CH)], pos_v.at[j % NPB], psem[j % NPB]
            )

        def start_store(u):
            j, b = divmod(u, B)
            return pltpu.async_copy(
                rows_v.at[u % NB],
                out_hbm.at[b, pl.ds(s0 + j * CH, CH)],
                ssem[u % NB],
            )

        poss = {j: start_pos(j) for j in range(min(NPB, NCHUNK))}
        gathers = {u: start_gather(u) for u in range(NB)}
        stores = {}
        for jg in range(NCHUNK):
            nbs = [(jg * B + b) % NB for b in range(B)]
            pb = jg % NPB
            for b in range(B):
                gathers.pop(jg * B + b).wait()
            poss.pop(jg).wait()

            def add_row(r, _, nbs=nbs, pb=pb):
                loads, sums = {}, {}

                def emit_loads(l):
                    sl = pl.ds(l * L, L)
                    loads[l] = (
                        pos_v[pb, r, sl],
                        [rows_v[nb, r, sl] for nb in nbs],
                    )

                def emit_adds(l):
                    p, rws = loads.pop(l)
                    sums[l] = [rw + p for rw in rws]

                def emit_stores(l):
                    sl = pl.ds(l * L, L)
                    for nb, v in zip(nbs, sums.pop(l)):
                        rows_v[nb, r, sl] = v

                emit_loads(0)
                emit_loads(1)
                emit_adds(0)
                for l in range(NL):
                    if l + 2 < NL:
                        emit_loads(l + 2)
                    if l + 1 < NL:
                        emit_adds(l + 1)
                    emit_stores(l)
                return 0

            lax.fori_loop(0, CH, add_row, 0)

            if jg + NPB < NCHUNK:
                poss[jg + NPB] = start_pos(jg + NPB)

            for b in range(B):
                u = jg * B + b
                stores[u] = start_store(u)
                nu = u + B
                if NB <= nu < NU:
                    blocker = nu - NB
                    if blocker in stores:
                        stores.pop(blocker).wait()
                    gathers[nu] = start_gather(nu)
        for st in stores.values():
            st.wait()

    return emb_kernel


def _chunk_major_tokens(tokens):
    t = tokens.reshape(B, NW, NCHUNK, CH)
    return t.transpose(1, 2, 0, 3).reshape(NW, NU * CH)


def kernel(tokens, token_embedding, position_embedding):
    emb = _make_kernel()
    tok_perm = _chunk_major_tokens(tokens.astype(jnp.int32))
    return emb(tok_perm, token_embedding, position_embedding)


# speedup vs baseline: 1.0324x; 1.0324x over previous
import functools

import jax
import jax.numpy as jnp
from jax import lax
from jax.experimental import pallas as pl
from jax.experimental.pallas import tpu as pltpu
from jax.experimental.pallas import tpu_sc as plsc

B = 4
S = 2048
D = 1024
L = 16

NC = 2
NS = 16
NW = NC * NS
S_PER_W = S // NW
CH = 16
NCHUNK = S_PER_W // CH
NU = B * NCHUNK
NB = 3


def _make_kernel():
    mesh = plsc.VectorSubcoreMesh(core_axis_name="c", subcore_axis_name="s")

    @functools.partial(
        pl.kernel,
        mesh=mesh,
        out_type=jax.ShapeDtypeStruct((B, S, D), jnp.float32),
        scratch_types=[
            pltpu.VMEM((B * S_PER_W,), jnp.int32),
            pltpu.VMEM((S_PER_W, D), jnp.float32),
            pltpu.VMEM((NB, CH, D), jnp.float32),
        ]
        + [pltpu.SemaphoreType.DMA] * (2 * NB + 1),
    )
    def emb_kernel(tok_hbm, tab_hbm, pos_hbm, out_hbm, idx_v, pos_v, rows_v, *sems):
        gsem, ssem, psem = sems[:NB], sems[NB : 2 * NB], sems[2 * NB]
        wid = lax.axis_index("s") * NC + lax.axis_index("c")
        s0 = wid * S_PER_W

        pltpu.sync_copy(tok_hbm.at[wid], idx_v)
        pos_cp = pltpu.async_copy(pos_hbm.at[pl.ds(s0, S_PER_W)], pos_v, psem)

        def start_gather(u):
            return pltpu.async_copy(
                tab_hbm.at[idx_v.at[pl.ds(u * CH, CH)]],
                rows_v.at[u % NB],
                gsem[u % NB],
            )

        gathers = {u: start_gather(u) for u in range(NB - 1)}
        pos_cp.wait()
        stores = {}
        for u in range(NU):
            b, j = divmod(u, NCHUNK)
            nb = u % NB
            gathers.pop(u).wait()
            nxt = u + NB - 1
            if nxt < NU:
                if nxt % NB in stores:
                    stores.pop(nxt % NB).wait()
                gathers[nxt] = start_gather(nxt)

            G = 8
            NL = D // L

            def add_row(r, _, j=j, nb=nb):
                vals = [pos_v[j * CH + r, pl.ds(l * L, L)] for l in range(G)]
                for l in range(NL):
                    if l + G < NL:
                        vals.append(pos_v[j * CH + r, pl.ds((l + G) * L, L)])
                    plsc.addupdate(rows_v.at[nb, r, pl.ds(l * L, L)], vals[l])
                return 0

            lax.fori_loop(0, CH, add_row, 0)

            stores[nb] = pltpu.async_copy(
                rows_v.at[nb],
                out_hbm.at[b, pl.ds(s0 + j * CH, CH)],
                ssem[nb],
            )
        for st in stores.values():
            st.wait()

    return emb_kernel


def kernel(tokens, token_embedding, position_embedding):
    emb = _make_kernel()
    tok_wm = (
        tokens.astype(jnp.int32)
        .reshape(B, NW, S_PER_W)
        .transpose(1, 0, 2)
        .reshape(NW, B * S_PER_W)
    )
    return emb(tok_wm, token_embedding, position_embedding)
